# Initial kernel scaffold; baseline (speedup 1.0000x reference)
import jax
import jax.numpy as jnp
from jax.experimental import pallas as pl


def kernel(x1, x2, conv_w, conv_b, on_w1, on_b1, on_gamma, on_beta, on_w2, on_b2, pr_w1, pr_b1, pr_gamma, pr_beta, pr_w2, pr_b2, tg_w1, tg_b1, tg_gamma, tg_beta, tg_w2, tg_b2):
    raise NotImplementedError("write your pallas kernel here")


# speedup vs baseline: 1.0482x; 1.0482x over previous
import jax
import jax.numpy as jnp
from jax.experimental import pallas as pl
from jax.experimental.pallas import tpu as pltpu

_BN_EPS = 1e-5
_NORM_EPS = 1e-12
_VMEM_LIMIT = 48 * 1024 * 1024


def _conv_gap_body(p_ref, w_ref, o_ref, *, img_tile, hw):
    w = w_ref[...]
    inv = 1.0 / hw
    for i in range(img_tile):
        y = jnp.dot(p_ref[i], w, preferred_element_type=jnp.float32)
        y = jnp.maximum(y, 0.0)
        s = jnp.sum(y, axis=0, keepdims=True) * inv
        o_ref[pl.ds(i, 1), :] = s.astype(o_ref.dtype)


def _conv_gap(patches, w_ext, *, img_tile=8):
    BB, HW, K = patches.shape
    F = w_ext.shape[1]
    return pl.pallas_call(
        lambda p, w, o: _conv_gap_body(p, w, o, img_tile=img_tile, hw=HW),
        out_shape=jax.ShapeDtypeStruct((BB, F), jnp.bfloat16),
        grid=(BB // img_tile,),
        in_specs=[
            pl.BlockSpec((img_tile, HW, K), lambda b: (b, 0, 0)),
            pl.BlockSpec((K, F), lambda b: (0, 0)),
        ],
        out_specs=pl.BlockSpec((img_tile, F), lambda b: (b, 0)),
        compiler_params=pltpu.CompilerParams(
            dimension_semantics=("parallel",),
            vmem_limit_bytes=_VMEM_LIMIT),
    )(patches, w_ext)


def _heads_body(f1, f2,
                ow1, ob1, og, obt, ow2, ob2,
                pw1, pb1, pg, pbt, pw2, pb2,
                tw1, tb1, tg, tbt, tw2, tb2,
                o_ref):
    def head(x, w1, b1, g, bt, w2, b2):
        pre = jnp.dot(x, w1[...], preferred_element_type=jnp.float32) + b1[...]
        mu = jnp.mean(pre, axis=0, keepdims=True)
        d = pre - mu
        var = jnp.mean(d * d, axis=0, keepdims=True)
        act = jnp.maximum(d * jax.lax.rsqrt(var + _BN_EPS) * g[...] + bt[...],
                          0.0)
        return jnp.dot(act.astype(w2.dtype), w2[...],
                       preferred_element_type=jnp.float32) + b2[...]

    z1 = head(f1[...], ow1, ob1, og, obt, ow2, ob2)
    z2 = head(f2[...], tw1, tb1, tg, tbt, tw2, tb2)
    q = head(z1.astype(pw1.dtype), pw1, pb1, pg, pbt, pw2, pb2)
    inv1 = jax.lax.rsqrt(jnp.maximum(
        jnp.sum(q * q, axis=-1, keepdims=True), _NORM_EPS * _NORM_EPS))
    inv2 = jax.lax.rsqrt(jnp.maximum(
        jnp.sum(z2 * z2, axis=-1, keepdims=True), _NORM_EPS * _NORM_EPS))
    sim = jnp.sum((q * inv1) * (z2 * inv2), axis=-1)
    o_ref[0] = 2.0 - 2.0 * (jnp.sum(sim) / q.shape[0])


def _heads_loss(f1, f2, args):
    def full(a):
        nd = a.ndim
        return pl.BlockSpec(a.shape, lambda _nd=nd: (0,) * _nd)

    ops = [f1, f2] + list(args)
    out = pl.pallas_call(
        _heads_body,
        out_shape=jax.ShapeDtypeStruct((1,), jnp.float32),
        grid=(),
        in_specs=[full(a) for a in ops],
        out_specs=pl.BlockSpec(memory_space=pltpu.MemorySpace.SMEM),
        compiler_params=pltpu.CompilerParams(
            vmem_limit_bytes=_VMEM_LIMIT),
    )(*ops)
    return out[0]


def _im2col_ext(x_nchw):
    x = jnp.transpose(x_nchw, (0, 2, 3, 1))
    B, H, W, C = x.shape
    xp = jnp.pad(x, ((0, 0), (1, 1), (1, 1), (0, 0)))
    cols = [xp[:, dh:dh + H, dw:dw + W, :] for dh in range(3) for dw in range(3)]
    cols.append(jnp.ones((B, H, W, 2), x.dtype))
    p = jnp.concatenate(cols, axis=-1)
    return p.reshape(B, H * W, 9 * C + 2).astype(jnp.bfloat16)


def kernel(x1, x2, conv_w, conv_b,
           on_w1, on_b1, on_gamma, on_beta, on_w2, on_b2,
           pr_w1, pr_b1, pr_gamma, pr_beta, pr_w2, pr_b2,
           tg_w1, tg_b1, tg_gamma, tg_beta, tg_w2, tg_b2):
    B = x1.shape[0]
    patches = jnp.concatenate([_im2col_ext(x1), _im2col_ext(x2)], axis=0)
    b_hi = conv_b.astype(jnp.bfloat16)
    b_lo = (conv_b - b_hi.astype(jnp.float32)).astype(jnp.bfloat16)
    w_ext = jnp.concatenate([conv_w, b_hi, b_lo], axis=0)

    f = _conv_gap(patches, w_ext)
    f1, f2 = f[:B], f[B:]
    return _heads_loss(f1, f2, [
        on_w1, on_b1, on_gamma, on_beta, on_w2, on_b2,
        pr_w1, pr_b1, pr_gamma, pr_beta, pr_w2, pr_b2,
        tg_w1, tg_b1, tg_gamma, tg_beta, tg_w2, tg_b2])
